# baseline (speedup 1.0000x reference)
import functools
import math

import jax
import jax.numpy as jnp
from jax import lax
from jax.experimental import pallas as pl
from jax.experimental.pallas import tpu as pltpu
from jax.experimental.pallas import tpu_sc as plsc

_S = 2048
_D = 768
_F = 3072
_H = 12
_DH = 64
_E = 8
_C = 512
_EPS = 1e-5
_RB = 256
_TRASH = _E * _C
_BUF_ROWS = _E * _C + _C

_NW = 32
_TPW = _S // _NW


def _ln(x, w, b):
    m = jnp.mean(x, axis=-1, keepdims=True)
    v = jnp.mean((x - m) ** 2, axis=-1, keepdims=True)
    return (x - m) / jnp.sqrt(v + _EPS) * w + b


def _dotT(a, b):
    return lax.dot_general(a.astype(jnp.bfloat16), b.astype(jnp.bfloat16),
                           (((1,), (1,)), ((), ())),
                           preferred_element_type=jnp.float32)


def _dot(a, b):
    return lax.dot_general(a.astype(jnp.bfloat16), b.astype(jnp.bfloat16),
                           (((1,), (0,)), ((), ())),
                           preferred_element_type=jnp.float32)


def _qkv_body(x_ref, w_ref, b_ref, lw_ref, lb_ref, qkv_ref):
    r = pl.program_id(0)
    h = _ln(x_ref[...], lw_ref[...], lb_ref[...])
    lane = lax.broadcasted_iota(jnp.int32, (_RB, _D), 1)
    evenl = (lane % 2) == 0
    sw = jnp.where(evenl, -jnp.roll(h, -1, axis=1), jnp.roll(h, 1, axis=1))
    pos = (r * _RB + lax.broadcasted_iota(jnp.int32, (_RB, _D), 0)).astype(jnp.float32)
    lane = lax.broadcasted_iota(jnp.int32, (_RB, _D), 1)
    j = (lane // 2).astype(jnp.float32)
    inv = jnp.exp(j * (-math.log(10000.0) / (_D // 2)))
    ang = pos * inv
    hr = h * jnp.cos(ang) + sw * jnp.sin(ang)
    qkv_ref[...] = _dot(hr, w_ref[...]) + b_ref[...]


def _attn_body(q_ref, k_ref, v_ref, o_ref):
    q2 = q_ref[...]
    k2 = k_ref[...]
    v2 = v_ref[...]
    outs = []
    scale = 1.0 / math.sqrt(_DH)
    for i in range(2):
        sl = slice(i * _DH, (i + 1) * _DH)
        q = q2[:, sl]
        k = k2[:, sl]
        s = _dotT(q, k) * scale
        qn = jnp.sqrt(jnp.sum(q * q, axis=-1, keepdims=True))
        kn = jnp.sqrt(jnp.max(jnp.sum(k * k, axis=-1, keepdims=True)))
        e = jnp.exp(s - qn * (kn * scale))
        denom = jnp.sum(e, axis=-1, keepdims=True)
        outs.append(_dot(e, v2[:, sl]) / denom)
    o_ref[...] = jnp.concatenate(outs, axis=-1)


def _post_body(ctx_ref, x_ref, w_ref, b_ref, lw_ref, lb_ref, rw_ref,
               attn_ref, h2_ref, lg_ref):
    proj = _dot(ctx_ref[...], w_ref[...]) + b_ref[...]
    a = x_ref[...] + proj
    attn_ref[...] = a
    h2 = _ln(a, lw_ref[...], lb_ref[...])
    h2_ref[...] = h2
    lg_ref[...] = lax.dot_general(h2, rw_ref[...], (((1,), (0,)), ((), ())))


def _route_body(lg_ref, sidx_ref, gidx_ref, w_ref):
    lg = lg_ref[...]
    mx = jnp.max(lg, axis=-1, keepdims=True)
    e = jnp.exp(lg - mx)
    emx = jnp.max(e, axis=-1, keepdims=True)
    ie = lax.broadcasted_iota(jnp.int32, (_S, _E), 1)
    top = jnp.min(jnp.where(e == emx, ie, _E), axis=-1, keepdims=True)
    onehot = (ie == top).astype(jnp.float32)
    ri = lax.broadcasted_iota(jnp.int32, (_S, _S), 0)
    ci = lax.broadcasted_iota(jnp.int32, (_S, _S), 1)
    tri = (ri >= ci).astype(jnp.float32)
    posm = lax.dot_general(tri, onehot, (((1,), (0,)), ((), ())))
    pos = jnp.sum(posm * onehot, axis=-1, keepdims=True)
    prob_top = emx / jnp.sum(e, axis=-1, keepdims=True)
    kept = pos <= float(_C)
    slot = top * _C + pos.astype(jnp.int32) - 1
    sidx_ref[...] = jnp.where(kept, slot, _TRASH)
    gidx_ref[...] = jnp.where(kept, slot, 0)
    w_ref[...] = jnp.where(kept, prob_top, 0.0)


def _ffn_body(buf_ref, wi_ref, wo_ref, eo_ref):
    xb = buf_ref[...]
    hid = _dot(xb, wi_ref[0])
    a = hid[:, : _F // 2]
    b = hid[:, _F // 2:]
    act = a * (1.0 / (1.0 + jnp.exp(-a))) * b
    eo_ref[...] = _dot(act, wo_ref[0])


def _combine_body(attn_ref, moe_ref, w_ref, out_ref):
    w = w_ref[...]
    out_ref[...] = attn_ref[...] + jnp.where(w > 0.0, w * moe_ref[...], 0.0)


def _sc_dispatch_body(h2_hbm, sidx_hbm, buf_hbm, idx_v, rows_v, sem):
    wid = lax.axis_index("s") * 2 + lax.axis_index("c")
    base = wid * _TPW
    pltpu.sync_copy(sidx_hbm.at[pl.ds(base, _TPW)], idx_v)
    pltpu.sync_copy(h2_hbm.at[pl.ds(base, _TPW)], rows_v)
    pltpu.async_copy(rows_v, buf_hbm.at[idx_v], sem).wait()


def _sc_combine_body(eo_hbm, gidx_hbm, out_hbm, idx_v, rows_v, sem):
    wid = lax.axis_index("s") * 2 + lax.axis_index("c")
    base = wid * _TPW
    pltpu.sync_copy(gidx_hbm.at[pl.ds(base, _TPW)], idx_v)
    pltpu.async_copy(eo_hbm.at[idx_v], rows_v, sem).wait()
    pltpu.sync_copy(rows_v, out_hbm.at[pl.ds(base, _TPW)])


def _sc_mesh():
    return plsc.VectorSubcoreMesh(core_axis_name="c", subcore_axis_name="s")


def kernel(x, ln1_w, ln1_b, in_proj_w, in_proj_b, out_proj_w, out_proj_b,
           ln2_w, ln2_b, router_w, wi, wo):
    B, S, D = x.shape
    f32 = jnp.float32
    x2 = x.reshape(S, D)
    row2 = lambda t: t.reshape(1, -1)

    nrb = S // _RB
    full = lambda shape: pl.BlockSpec(shape, lambda r: (0,) * len(shape))

    qkv = pl.pallas_call(
        _qkv_body,
        grid=(nrb,),
        in_specs=[
            pl.BlockSpec((_RB, D), lambda r: (r, 0)),
            full((D, 3 * D)),
            full((1, 3 * D)),
            full((1, D)),
            full((1, D)),
        ],
        out_specs=pl.BlockSpec((_RB, 3 * D), lambda r: (r, 0)),
        out_shape=jax.ShapeDtypeStruct((S, 3 * D), f32),
    )(x2, in_proj_w.T, row2(in_proj_b), row2(ln1_w), row2(ln1_b))

    rows_attn = 2
    rblk = S // rows_attn
    npair = _H // 2
    ctx2 = pl.pallas_call(
        _attn_body,
        grid=(npair, rows_attn),
        in_specs=[
            pl.BlockSpec((rblk, 2 * _DH), lambda p, r: (r, p)),
            pl.BlockSpec((S, 2 * _DH), lambda p, r: (0, npair + p)),
            pl.BlockSpec((S, 2 * _DH), lambda p, r: (0, 2 * npair + p)),
        ],
        out_specs=pl.BlockSpec((rblk, 2 * _DH), lambda p, r: (r, p)),
        out_shape=jax.ShapeDtypeStruct((S, D), f32),
    )(qkv, qkv, qkv)

    attn_out, h2, logits = pl.pallas_call(
        _post_body,
        grid=(nrb,),
        in_specs=[
            pl.BlockSpec((_RB, D), lambda r: (r, 0)),
            pl.BlockSpec((_RB, D), lambda r: (r, 0)),
            full((D, D)),
            full((1, D)),
            full((1, D)),
            full((1, D)),
            full((D, _E)),
        ],
        out_specs=[
            pl.BlockSpec((_RB, D), lambda r: (r, 0)),
            pl.BlockSpec((_RB, D), lambda r: (r, 0)),
            pl.BlockSpec((_RB, _E), lambda r: (r, 0)),
        ],
        out_shape=[
            jax.ShapeDtypeStruct((S, D), f32),
            jax.ShapeDtypeStruct((S, D), f32),
            jax.ShapeDtypeStruct((S, _E), f32),
        ],
    )(ctx2, x2, out_proj_w.T, row2(out_proj_b), row2(ln2_w), row2(ln2_b), router_w)

    sidx2, gidx2, w2 = pl.pallas_call(
        _route_body,
        grid=(1,),
        in_specs=[full((S, _E))],
        out_specs=[full((S, 1)), full((S, 1)), full((S, 1))],
        out_shape=[
            jax.ShapeDtypeStruct((S, 1), jnp.int32),
            jax.ShapeDtypeStruct((S, 1), jnp.int32),
            jax.ShapeDtypeStruct((S, 1), f32),
        ],
    )(logits)
    sidx = sidx2.reshape(S)
    gidx = gidx2.reshape(S)

    dispatch = functools.partial(
        pl.kernel,
        mesh=_sc_mesh(),
        out_type=jax.ShapeDtypeStruct((_BUF_ROWS, D), f32),
        scratch_types=[
            pltpu.VMEM((_TPW,), jnp.int32),
            pltpu.VMEM((_TPW, D), f32),
            pltpu.SemaphoreType.DMA,
        ],
    )(_sc_dispatch_body)
    buf = dispatch(h2, sidx)

    eo = pl.pallas_call(
        _ffn_body,
        grid=(_E,),
        in_specs=[
            pl.BlockSpec((_C, D), lambda e: (e, 0)),
            pl.BlockSpec((1, D, _F), lambda e: (e, 0, 0)),
            pl.BlockSpec((1, _F // 2, D), lambda e: (e, 0, 0)),
        ],
        out_specs=pl.BlockSpec((_C, D), lambda e: (e, 0)),
        out_shape=jax.ShapeDtypeStruct((_E * _C, D), f32),
    )(buf, wi, wo)

    combine = functools.partial(
        pl.kernel,
        mesh=_sc_mesh(),
        out_type=jax.ShapeDtypeStruct((S, D), f32),
        scratch_types=[
            pltpu.VMEM((_TPW,), jnp.int32),
            pltpu.VMEM((_TPW, D), f32),
            pltpu.SemaphoreType.DMA,
        ],
    )(_sc_combine_body)
    moe_rows = combine(eo, gidx)

    out2 = pl.pallas_call(
        _combine_body,
        grid=(nrb,),
        in_specs=[
            pl.BlockSpec((_RB, D), lambda r: (r, 0)),
            pl.BlockSpec((_RB, D), lambda r: (r, 0)),
            pl.BlockSpec((_RB, 1), lambda r: (r, 0)),
        ],
        out_specs=pl.BlockSpec((_RB, D), lambda r: (r, 0)),
        out_shape=jax.ShapeDtypeStruct((S, D), f32),
    )(attn_out, moe_rows, w2)

    return out2.reshape(B, S, D), logits.reshape(B, S, _E)


# speedup vs baseline: 1.0323x; 1.0323x over previous
import functools
import math

import jax
import jax.numpy as jnp
from jax import lax
from jax.experimental import pallas as pl
from jax.experimental.pallas import tpu as pltpu
from jax.experimental.pallas import tpu_sc as plsc

_S = 2048
_D = 768
_F = 3072
_H = 12
_DH = 64
_E = 8
_C = 512
_EPS = 1e-5
_RB = 512
_TRASH = _E * _C
_BUF_ROWS = _E * _C + _C

_NW = 32
_TPW = _S // _NW


def _ln(x, w, b):
    m = jnp.mean(x, axis=-1, keepdims=True)
    v = jnp.mean((x - m) ** 2, axis=-1, keepdims=True)
    return (x - m) / jnp.sqrt(v + _EPS) * w + b


def _dotT(a, b):
    return lax.dot_general(a.astype(jnp.bfloat16), b.astype(jnp.bfloat16),
                           (((1,), (1,)), ((), ())),
                           preferred_element_type=jnp.float32)


def _dot(a, b):
    return lax.dot_general(a.astype(jnp.bfloat16), b.astype(jnp.bfloat16),
                           (((1,), (0,)), ((), ())),
                           preferred_element_type=jnp.float32)


def _qkv_body(x_ref, w_ref, b_ref, lw_ref, lb_ref, qkv_ref):
    r = pl.program_id(0)
    h = _ln(x_ref[...], lw_ref[...], lb_ref[...])
    lane = lax.broadcasted_iota(jnp.int32, (_RB, _D), 1)
    evenl = (lane % 2) == 0
    sw = jnp.where(evenl, -jnp.roll(h, -1, axis=1), jnp.roll(h, 1, axis=1))
    pos = (r * _RB + lax.broadcasted_iota(jnp.int32, (_RB, _D), 0)).astype(jnp.float32)
    lane = lax.broadcasted_iota(jnp.int32, (_RB, _D), 1)
    j = (lane // 2).astype(jnp.float32)
    inv = jnp.exp(j * (-math.log(10000.0) / (_D // 2)))
    ang = pos * inv
    hr = h * jnp.cos(ang) + sw * jnp.sin(ang)
    qkv_ref[...] = _dotT(hr, w_ref[...]) + b_ref[...]


def _attn_body(q_ref, k_ref, v_ref, o_ref):
    q2 = q_ref[...]
    k2 = k_ref[...]
    v2 = v_ref[...]
    outs = []
    scale = 1.0 / math.sqrt(_DH)
    for i in range(2):
        sl = slice(i * _DH, (i + 1) * _DH)
        q = q2[:, sl]
        k = k2[:, sl]
        s = _dotT(q, k) * scale
        qn = jnp.sqrt(jnp.sum(q * q, axis=-1, keepdims=True))
        kn = jnp.sqrt(jnp.max(jnp.sum(k * k, axis=-1, keepdims=True)))
        e = jnp.exp(s - qn * (kn * scale))
        denom = jnp.sum(e, axis=-1, keepdims=True)
        outs.append(_dot(e, v2[:, sl]) / denom)
    o_ref[...] = jnp.concatenate(outs, axis=-1)


def _post_body(ctx_ref, x_ref, w_ref, b_ref, lw_ref, lb_ref, rw_ref,
               attn_ref, h2_ref, lg_ref):
    proj = _dotT(ctx_ref[...], w_ref[...]) + b_ref[...]
    a = x_ref[...] + proj
    attn_ref[...] = a
    h2 = _ln(a, lw_ref[...], lb_ref[...])
    h2_ref[...] = h2
    lg_ref[...] = lax.dot_general(h2, rw_ref[...], (((1,), (0,)), ((), ())))


def _route_body(lg_ref, sidx_ref, gidx_ref, w_ref):
    lg = lg_ref[...]
    mx = jnp.max(lg, axis=-1, keepdims=True)
    e = jnp.exp(lg - mx)
    emx = jnp.max(e, axis=-1, keepdims=True)
    ie = lax.broadcasted_iota(jnp.int32, (_S, _E), 1)
    top = jnp.min(jnp.where(e == emx, ie, _E), axis=-1, keepdims=True)
    onehot = (ie == top).astype(jnp.float32)
    ri = lax.broadcasted_iota(jnp.int32, (_S, _S), 0)
    ci = lax.broadcasted_iota(jnp.int32, (_S, _S), 1)
    tri = (ri >= ci).astype(jnp.float32)
    posm = lax.dot_general(tri, onehot, (((1,), (0,)), ((), ())))
    pos = jnp.sum(posm * onehot, axis=-1, keepdims=True)
    prob_top = emx / jnp.sum(e, axis=-1, keepdims=True)
    kept = pos <= float(_C)
    slot = top * _C + pos.astype(jnp.int32) - 1
    sidx_ref[...] = jnp.where(kept, slot, _TRASH)
    gidx_ref[...] = jnp.where(kept, slot, 0)
    w_ref[...] = jnp.where(kept, prob_top, 0.0)


def _ffn_body(buf_ref, wi_ref, wo_ref, eo_ref):
    xb = buf_ref[...]
    hid = _dot(xb, wi_ref[0])
    a = hid[:, : _F // 2]
    b = hid[:, _F // 2:]
    act = a * (1.0 / (1.0 + jnp.exp(-a))) * b
    eo_ref[...] = _dot(act, wo_ref[0])


def _combine_body(attn_ref, moe_ref, w_ref, out_ref):
    w = w_ref[...]
    out_ref[...] = attn_ref[...] + jnp.where(w > 0.0, w * moe_ref[...], 0.0)


def _sc_dispatch_body(h2_hbm, sidx_hbm, buf_hbm, idx_v, rows_v, sem):
    wid = lax.axis_index("s") * 2 + lax.axis_index("c")
    base = wid * _TPW
    pltpu.sync_copy(sidx_hbm.at[pl.ds(base, _TPW)], idx_v)
    pltpu.sync_copy(h2_hbm.at[pl.ds(base, _TPW)], rows_v)
    pltpu.async_copy(rows_v, buf_hbm.at[idx_v], sem).wait()


def _sc_combine_body(eo_hbm, gidx_hbm, out_hbm, idx_v, rows_v, sem):
    wid = lax.axis_index("s") * 2 + lax.axis_index("c")
    base = wid * _TPW
    pltpu.sync_copy(gidx_hbm.at[pl.ds(base, _TPW)], idx_v)
    pltpu.async_copy(eo_hbm.at[idx_v], rows_v, sem).wait()
    pltpu.sync_copy(rows_v, out_hbm.at[pl.ds(base, _TPW)])


def _sc_mesh():
    return plsc.VectorSubcoreMesh(core_axis_name="c", subcore_axis_name="s")


def kernel(x, ln1_w, ln1_b, in_proj_w, in_proj_b, out_proj_w, out_proj_b,
           ln2_w, ln2_b, router_w, wi, wo):
    B, S, D = x.shape
    f32 = jnp.float32
    x2 = x.reshape(S, D)
    row2 = lambda t: t.reshape(1, -1)

    nrb = S // _RB
    full = lambda shape: pl.BlockSpec(shape, lambda r: (0,) * len(shape))

    qkv = pl.pallas_call(
        _qkv_body,
        grid=(nrb,),
        in_specs=[
            pl.BlockSpec((_RB, D), lambda r: (r, 0)),
            full((3 * D, D)),
            full((1, 3 * D)),
            full((1, D)),
            full((1, D)),
        ],
        out_specs=pl.BlockSpec((_RB, 3 * D), lambda r: (r, 0)),
        out_shape=jax.ShapeDtypeStruct((S, 3 * D), f32),
    )(x2, in_proj_w, row2(in_proj_b), row2(ln1_w), row2(ln1_b))

    rows_attn = 2
    rblk = S // rows_attn
    npair = _H // 2
    ctx2 = pl.pallas_call(
        _attn_body,
        grid=(npair, rows_attn),
        in_specs=[
            pl.BlockSpec((rblk, 2 * _DH), lambda p, r: (r, p)),
            pl.BlockSpec((S, 2 * _DH), lambda p, r: (0, npair + p)),
            pl.BlockSpec((S, 2 * _DH), lambda p, r: (0, 2 * npair + p)),
        ],
        out_specs=pl.BlockSpec((rblk, 2 * _DH), lambda p, r: (r, p)),
        out_shape=jax.ShapeDtypeStruct((S, D), f32),
    )(qkv, qkv, qkv)

    attn_out, h2, logits = pl.pallas_call(
        _post_body,
        grid=(nrb,),
        in_specs=[
            pl.BlockSpec((_RB, D), lambda r: (r, 0)),
            pl.BlockSpec((_RB, D), lambda r: (r, 0)),
            full((D, D)),
            full((1, D)),
            full((1, D)),
            full((1, D)),
            full((D, _E)),
        ],
        out_specs=[
            pl.BlockSpec((_RB, D), lambda r: (r, 0)),
            pl.BlockSpec((_RB, D), lambda r: (r, 0)),
            pl.BlockSpec((_RB, _E), lambda r: (r, 0)),
        ],
        out_shape=[
            jax.ShapeDtypeStruct((S, D), f32),
            jax.ShapeDtypeStruct((S, D), f32),
            jax.ShapeDtypeStruct((S, _E), f32),
        ],
    )(ctx2, x2, out_proj_w, row2(out_proj_b), row2(ln2_w), row2(ln2_b), router_w)

    sidx2, gidx2, w2 = pl.pallas_call(
        _route_body,
        grid=(1,),
        in_specs=[full((S, _E))],
        out_specs=[full((S, 1)), full((S, 1)), full((S, 1))],
        out_shape=[
            jax.ShapeDtypeStruct((S, 1), jnp.int32),
            jax.ShapeDtypeStruct((S, 1), jnp.int32),
            jax.ShapeDtypeStruct((S, 1), f32),
        ],
    )(logits)
    sidx = sidx2.reshape(S)
    gidx = gidx2.reshape(S)

    dispatch = functools.partial(
        pl.kernel,
        mesh=_sc_mesh(),
        out_type=jax.ShapeDtypeStruct((_BUF_ROWS, D), f32),
        scratch_types=[
            pltpu.VMEM((_TPW,), jnp.int32),
            pltpu.VMEM((_TPW, D), f32),
            pltpu.SemaphoreType.DMA,
        ],
    )(_sc_dispatch_body)
    buf = dispatch(h2, sidx)

    eo = pl.pallas_call(
        _ffn_body,
        grid=(_E,),
        in_specs=[
            pl.BlockSpec((_C, D), lambda e: (e, 0)),
            pl.BlockSpec((1, D, _F), lambda e: (e, 0, 0)),
            pl.BlockSpec((1, _F // 2, D), lambda e: (e, 0, 0)),
        ],
        out_specs=pl.BlockSpec((_C, D), lambda e: (e, 0)),
        out_shape=jax.ShapeDtypeStruct((_E * _C, D), f32),
    )(buf, wi, wo)

    combine = functools.partial(
        pl.kernel,
        mesh=_sc_mesh(),
        out_type=jax.ShapeDtypeStruct((S, D), f32),
        scratch_types=[
            pltpu.VMEM((_TPW,), jnp.int32),
            pltpu.VMEM((_TPW, D), f32),
            pltpu.SemaphoreType.DMA,
        ],
    )(_sc_combine_body)
    moe_rows = combine(eo, gidx)

    out2 = pl.pallas_call(
        _combine_body,
        grid=(nrb,),
        in_specs=[
            pl.BlockSpec((_RB, D), lambda r: (r, 0)),
            pl.BlockSpec((_RB, D), lambda r: (r, 0)),
            pl.BlockSpec((_RB, 1), lambda r: (r, 0)),
        ],
        out_specs=pl.BlockSpec((_RB, D), lambda r: (r, 0)),
        out_shape=jax.ShapeDtypeStruct((S, D), f32),
    )(attn_out, moe_rows, w2)

    return out2.reshape(B, S, D), logits.reshape(B, S, _E)


# speedup vs baseline: 1.0328x; 1.0005x over previous
import functools
import math

import jax
import jax.numpy as jnp
from jax import lax
from jax.experimental import pallas as pl
from jax.experimental.pallas import tpu as pltpu
from jax.experimental.pallas import tpu_sc as plsc

_S = 2048
_D = 768
_F = 3072
_H = 12
_DH = 64
_E = 8
_C = 512
_EPS = 1e-5
_RB = 1024
_TRASH = _E * _C
_BUF_ROWS = _E * _C + _C

_NW = 32
_TPW = _S // _NW


def _ln(x, w, b):
    m = jnp.mean(x, axis=-1, keepdims=True)
    v = jnp.mean((x - m) ** 2, axis=-1, keepdims=True)
    return (x - m) / jnp.sqrt(v + _EPS) * w + b


def _dotT(a, b):
    return lax.dot_general(a.astype(jnp.bfloat16), b.astype(jnp.bfloat16),
                           (((1,), (1,)), ((), ())),
                           preferred_element_type=jnp.float32)


def _dot(a, b):
    return lax.dot_general(a.astype(jnp.bfloat16), b.astype(jnp.bfloat16),
                           (((1,), (0,)), ((), ())),
                           preferred_element_type=jnp.float32)


def _qkv_body(x_ref, w_ref, b_ref, lw_ref, lb_ref, qkv_ref):
    r = pl.program_id(0)
    h = _ln(x_ref[...], lw_ref[...], lb_ref[...])
    lane = lax.broadcasted_iota(jnp.int32, (_RB, _D), 1)
    evenl = (lane % 2) == 0
    sw = jnp.where(evenl, -jnp.roll(h, -1, axis=1), jnp.roll(h, 1, axis=1))
    pos = (r * _RB + lax.broadcasted_iota(jnp.int32, (_RB, _D), 0)).astype(jnp.float32)
    lane = lax.broadcasted_iota(jnp.int32, (_RB, _D), 1)
    j = (lane // 2).astype(jnp.float32)
    inv = jnp.exp(j * (-math.log(10000.0) / (_D // 2)))
    ang = pos * inv
    hr = h * jnp.cos(ang) + sw * jnp.sin(ang)
    qkv_ref[...] = _dotT(hr, w_ref[...]) + b_ref[...]


def _attn_body(q_ref, k_ref, v_ref, o_ref):
    q2 = q_ref[...]
    k2 = k_ref[...]
    v2 = v_ref[...]
    outs = []
    scale = 1.0 / math.sqrt(_DH)
    for i in range(2):
        sl = slice(i * _DH, (i + 1) * _DH)
        q = q2[:, sl]
        k = k2[:, sl]
        s = _dotT(q, k) * scale
        qn = jnp.sqrt(jnp.sum(q * q, axis=-1, keepdims=True))
        kn = jnp.sqrt(jnp.max(jnp.sum(k * k, axis=-1, keepdims=True)))
        e = jnp.exp(s - qn * (kn * scale))
        denom = jnp.sum(e, axis=-1, keepdims=True)
        outs.append(_dot(e, v2[:, sl]) / denom)
    o_ref[...] = jnp.concatenate(outs, axis=-1)


def _post_body(ctx_ref, x_ref, w_ref, b_ref, lw_ref, lb_ref, rw_ref,
               attn_ref, h2_ref, lg_ref):
    proj = _dotT(ctx_ref[...], w_ref[...]) + b_ref[...]
    a = x_ref[...] + proj
    attn_ref[...] = a
    h2 = _ln(a, lw_ref[...], lb_ref[...])
    h2_ref[...] = h2
    lg_ref[...] = lax.dot_general(h2, rw_ref[...], (((1,), (0,)), ((), ())))


def _route_body(lg_ref, sidx_ref, gidx_ref, w_ref):
    lg = lg_ref[...]
    mx = jnp.max(lg, axis=-1, keepdims=True)
    e = jnp.exp(lg - mx)
    emx = jnp.max(e, axis=-1, keepdims=True)
    ie = lax.broadcasted_iota(jnp.int32, (_S, _E), 1)
    top = jnp.min(jnp.where(e == emx, ie, _E), axis=-1, keepdims=True)
    onehot = (ie == top).astype(jnp.float32)
    ri = lax.broadcasted_iota(jnp.int32, (_S, _S), 0)
    ci = lax.broadcasted_iota(jnp.int32, (_S, _S), 1)
    tri = (ri >= ci).astype(jnp.float32)
    posm = lax.dot_general(tri, onehot, (((1,), (0,)), ((), ())))
    pos = jnp.sum(posm * onehot, axis=-1, keepdims=True)
    prob_top = emx / jnp.sum(e, axis=-1, keepdims=True)
    kept = pos <= float(_C)
    slot = top * _C + pos.astype(jnp.int32) - 1
    sidx_ref[...] = jnp.where(kept, slot, _TRASH)
    gidx_ref[...] = jnp.where(kept, slot, 0)
    w_ref[...] = jnp.where(kept, prob_top, 0.0)


def _ffn_body(buf_ref, wi_ref, wo_ref, eo_ref):
    xb = buf_ref[...]
    hid = _dot(xb, wi_ref[0])
    a = hid[:, : _F // 2]
    b = hid[:, _F // 2:]
    act = a * (1.0 / (1.0 + jnp.exp(-a))) * b
    eo_ref[...] = _dot(act, wo_ref[0])


def _combine_body(attn_ref, moe_ref, w_ref, out_ref):
    w = w_ref[...]
    out_ref[...] = attn_ref[...] + jnp.where(w > 0.0, w * moe_ref[...], 0.0)


def _sc_dispatch_body(h2_hbm, sidx_hbm, buf_hbm, idx_v, rows_v, sem):
    wid = lax.axis_index("s") * 2 + lax.axis_index("c")
    base = wid * _TPW
    pltpu.sync_copy(sidx_hbm.at[pl.ds(base, _TPW)], idx_v)
    pltpu.sync_copy(h2_hbm.at[pl.ds(base, _TPW)], rows_v)
    pltpu.async_copy(rows_v, buf_hbm.at[idx_v], sem).wait()


def _sc_combine_body(eo_hbm, gidx_hbm, out_hbm, idx_v, rows_v, sem):
    wid = lax.axis_index("s") * 2 + lax.axis_index("c")
    base = wid * _TPW
    pltpu.sync_copy(gidx_hbm.at[pl.ds(base, _TPW)], idx_v)
    pltpu.async_copy(eo_hbm.at[idx_v], rows_v, sem).wait()
    pltpu.sync_copy(rows_v, out_hbm.at[pl.ds(base, _TPW)])


def _sc_mesh():
    return plsc.VectorSubcoreMesh(core_axis_name="c", subcore_axis_name="s")


def kernel(x, ln1_w, ln1_b, in_proj_w, in_proj_b, out_proj_w, out_proj_b,
           ln2_w, ln2_b, router_w, wi, wo):
    B, S, D = x.shape
    f32 = jnp.float32
    x2 = x.reshape(S, D)
    row2 = lambda t: t.reshape(1, -1)

    nrb = S // _RB
    full = lambda shape: pl.BlockSpec(shape, lambda r: (0,) * len(shape))

    qkv = pl.pallas_call(
        _qkv_body,
        grid=(nrb,),
        in_specs=[
            pl.BlockSpec((_RB, D), lambda r: (r, 0)),
            full((3 * D, D)),
            full((1, 3 * D)),
            full((1, D)),
            full((1, D)),
        ],
        out_specs=pl.BlockSpec((_RB, 3 * D), lambda r: (r, 0)),
        out_shape=jax.ShapeDtypeStruct((S, 3 * D), f32),
    )(x2, in_proj_w, row2(in_proj_b), row2(ln1_w), row2(ln1_b))

    rows_attn = 2
    rblk = S // rows_attn
    npair = _H // 2
    ctx2 = pl.pallas_call(
        _attn_body,
        grid=(npair, rows_attn),
        in_specs=[
            pl.BlockSpec((rblk, 2 * _DH), lambda p, r: (r, p)),
            pl.BlockSpec((S, 2 * _DH), lambda p, r: (0, npair + p)),
            pl.BlockSpec((S, 2 * _DH), lambda p, r: (0, 2 * npair + p)),
        ],
        out_specs=pl.BlockSpec((rblk, 2 * _DH), lambda p, r: (r, p)),
        out_shape=jax.ShapeDtypeStruct((S, D), f32),
    )(qkv, qkv, qkv)

    attn_out, h2, logits = pl.pallas_call(
        _post_body,
        grid=(nrb,),
        in_specs=[
            pl.BlockSpec((_RB, D), lambda r: (r, 0)),
            pl.BlockSpec((_RB, D), lambda r: (r, 0)),
            full((D, D)),
            full((1, D)),
            full((1, D)),
            full((1, D)),
            full((D, _E)),
        ],
        out_specs=[
            pl.BlockSpec((_RB, D), lambda r: (r, 0)),
            pl.BlockSpec((_RB, D), lambda r: (r, 0)),
            pl.BlockSpec((_RB, _E), lambda r: (r, 0)),
        ],
        out_shape=[
            jax.ShapeDtypeStruct((S, D), f32),
            jax.ShapeDtypeStruct((S, D), f32),
            jax.ShapeDtypeStruct((S, _E), f32),
        ],
    )(ctx2, x2, out_proj_w, row2(out_proj_b), row2(ln2_w), row2(ln2_b), router_w)

    sidx2, gidx2, w2 = pl.pallas_call(
        _route_body,
        grid=(1,),
        in_specs=[full((S, _E))],
        out_specs=[full((S, 1)), full((S, 1)), full((S, 1))],
        out_shape=[
            jax.ShapeDtypeStruct((S, 1), jnp.int32),
            jax.ShapeDtypeStruct((S, 1), jnp.int32),
            jax.ShapeDtypeStruct((S, 1), f32),
        ],
    )(logits)
    sidx = sidx2.reshape(S)
    gidx = gidx2.reshape(S)

    dispatch = functools.partial(
        pl.kernel,
        mesh=_sc_mesh(),
        out_type=jax.ShapeDtypeStruct((_BUF_ROWS, D), f32),
        scratch_types=[
            pltpu.VMEM((_TPW,), jnp.int32),
            pltpu.VMEM((_TPW, D), f32),
            pltpu.SemaphoreType.DMA,
        ],
    )(_sc_dispatch_body)
    buf = dispatch(h2, sidx)

    eo = pl.pallas_call(
        _ffn_body,
        grid=(_E,),
        in_specs=[
            pl.BlockSpec((_C, D), lambda e: (e, 0)),
            pl.BlockSpec((1, D, _F), lambda e: (e, 0, 0)),
            pl.BlockSpec((1, _F // 2, D), lambda e: (e, 0, 0)),
        ],
        out_specs=pl.BlockSpec((_C, D), lambda e: (e, 0)),
        out_shape=jax.ShapeDtypeStruct((_E * _C, D), f32),
    )(buf, wi, wo)

    combine = functools.partial(
        pl.kernel,
        mesh=_sc_mesh(),
        out_type=jax.ShapeDtypeStruct((S, D), f32),
        scratch_types=[
            pltpu.VMEM((_TPW,), jnp.int32),
            pltpu.VMEM((_TPW, D), f32),
            pltpu.SemaphoreType.DMA,
        ],
    )(_sc_combine_body)
    moe_rows = combine(eo, gidx)

    out2 = pl.pallas_call(
        _combine_body,
        grid=(nrb,),
        in_specs=[
            pl.BlockSpec((_RB, D), lambda r: (r, 0)),
            pl.BlockSpec((_RB, D), lambda r: (r, 0)),
            pl.BlockSpec((_RB, 1), lambda r: (r, 0)),
        ],
        out_specs=pl.BlockSpec((_RB, D), lambda r: (r, 0)),
        out_shape=jax.ShapeDtypeStruct((S, D), f32),
    )(attn_out, moe_rows, w2)

    return out2.reshape(B, S, D), logits.reshape(B, S, _E)
